# Initial kernel scaffold; baseline (speedup 1.0000x reference)
import jax
import jax.numpy as jnp
from jax.experimental import pallas as pl


def kernel(pred_logits, pred_polylines, tgt_labels, tgt_polylines):
    raise NotImplementedError("write your pallas kernel here")


# speedup vs baseline: 104.0675x; 104.0675x over previous
import jax
import jax.numpy as jnp
from jax import lax
from jax.experimental import pallas as pl
from jax.experimental.pallas import tpu as pltpu

_NC = 50
_PW = 5.0


def _body(xT_ref, ppT_ref, lab_ref, tp_ref, out_ref, lp_scr, poly_scr, c_scr):
    B, C, Q = xT_ref.shape
    T = lab_ref.shape[1]
    D = ppT_ref.shape[1]

    xT = xT_ref[...]
    m = jnp.max(xT, axis=1, keepdims=True)
    e = jnp.exp(xT - m)
    s = jnp.sum(e, axis=1, keepdims=True)
    logls = jnp.log(s)
    logp_bg = xT[:, _NC:_NC + 1, :] - m - logls
    ce_bg = jnp.sum(-logp_bg)

    labs = lab_ref[...]
    oh = (labs[:, :, None] ==
          lax.broadcasted_iota(jnp.int32, (B, T, C), 2)).astype(jnp.float32)
    for b in range(B):
        lp_scr[b] = lax.dot_general(oh[b], xT[b], (((1,), (0,)), ((), ())),
                                    preferred_element_type=jnp.float32)
    xl = lp_scr[...]
    c_scr[...] = -(jnp.exp(xl - m) / s)
    lp_scr[...] = xl - m - logls

    poly = jnp.zeros((B, T, Q), dtype=jnp.float32)
    tp = tp_ref[...]
    for d in range(D):
        a_d = ppT_ref[:, d:d + 1, :]
        b_d = tp[:, :, d:d + 1]
        poly = poly + jnp.abs(a_d - b_d)
    poly_scr[...] = poly
    c_scr[...] = c_scr[...] + _PW * poly

    logp_bg_v = logp_bg
    qi = lax.broadcasted_iota(jnp.int32, (B, 1, Q), 2)

    def gbody(t, carry):
        used, pacc, cacc = carry
        crow = c_scr[:, pl.ds(t, 1), :] + used
        mn = jnp.min(crow, axis=2, keepdims=True)
        cand = jnp.where(crow == mn, qi, Q)
        am = jnp.min(cand, axis=2, keepdims=True)
        sel = qi == am
        used = used + jnp.where(sel, jnp.float32(jnp.inf), 0.0)
        prow = poly_scr[:, pl.ds(t, 1), :]
        lrow = lp_scr[:, pl.ds(t, 1), :]
        pacc = pacc + jnp.sum(jnp.where(sel, prow, 0.0))
        cacc = cacc + jnp.sum(jnp.where(sel, logp_bg_v - lrow, 0.0))
        return used, pacc, cacc

    used0 = jnp.zeros((B, 1, Q), dtype=jnp.float32)
    _, pacc, cacc = lax.fori_loop(0, T, gbody, (used0, jnp.float32(0.0), jnp.float32(0.0)))

    loss_ce = (ce_bg + cacc) / jnp.float32(B * Q)
    loss_poly = pacc / jnp.float32(B * T)
    lane = lax.broadcasted_iota(jnp.int32, (1, 2), 1)
    out_ref[...] = jnp.where(lane == 0, loss_ce, loss_poly)


def kernel(pred_logits, pred_polylines, tgt_labels, tgt_polylines):
    B, Q, C = pred_logits.shape
    T = tgt_labels.shape[1]
    xT = jnp.transpose(pred_logits, (0, 2, 1))
    ppT = jnp.transpose(pred_polylines, (0, 2, 1))
    out = pl.pallas_call(
        _body,
        out_shape=jax.ShapeDtypeStruct((1, 2), jnp.float32),
        scratch_shapes=[
            pltpu.VMEM((B, T, Q), jnp.float32),
            pltpu.VMEM((B, T, Q), jnp.float32),
            pltpu.VMEM((B, T, Q), jnp.float32),
        ],
    )(xT, ppT, tgt_labels.astype(jnp.int32), tgt_polylines)
    return out.reshape(2)
